# Initial kernel scaffold; baseline (speedup 1.0000x reference)
import jax
import jax.numpy as jnp
from jax.experimental import pallas as pl


def kernel(x, word2vec_matrix):
    raise NotImplementedError("write your pallas kernel here")


# speedup vs baseline: 3.5493x; 3.5493x over previous
import functools

import jax
import jax.numpy as jnp
from jax import lax
from jax.experimental import pallas as pl
from jax.experimental.pallas import tpu as pltpu
from jax.experimental.pallas import tpu_sc as plsc

VOCAB = 100000
EMBED_DIM = 64
BATCH = 4096
SEQ_LEN = 200

NC = 2
NS = 16
NW = NC * NS

TOTAL = BATCH * SEQ_LEN
ROWS_PER_W = TOTAL // NW
CHUNK = 128
CHUNKS = ROWS_PER_W // CHUNK


@functools.partial(
    pl.kernel,
    out_type=jax.ShapeDtypeStruct((TOTAL, EMBED_DIM), jnp.float32),
    mesh=plsc.VectorSubcoreMesh(core_axis_name="c", subcore_axis_name="s"),
    compiler_params=pltpu.CompilerParams(use_tc_tiling_on_sc=False),
    scratch_types=[
        pltpu.VMEM((CHUNKS, CHUNK), jnp.int32),
        pltpu.VMEM((CHUNK, EMBED_DIM), jnp.float32),
        pltpu.SemaphoreType.DMA,
    ],
)
def _embed_lookup(idx_hbm, table_hbm, out_hbm, idx_v, rows_v, sem):
    wid = lax.axis_index("s") * NC + lax.axis_index("c")
    base = wid * ROWS_PER_W
    pltpu.sync_copy(idx_hbm.at[wid], idx_v)

    def step(c, carry):
        pltpu.async_copy(table_hbm.at[idx_v.at[c]], rows_v, sem).wait()
        pltpu.sync_copy(rows_v, out_hbm.at[pl.ds(base + c * CHUNK, CHUNK)])
        return carry

    lax.fori_loop(0, CHUNKS, step, 0)


def kernel(x, word2vec_matrix):
    idx = x.reshape(NW, CHUNKS, CHUNK).astype(jnp.int32)
    out = _embed_lookup(idx, word2vec_matrix)
    return out.reshape(BATCH, SEQ_LEN, EMBED_DIM)


# speedup vs baseline: 3.7781x; 1.0645x over previous
import functools

import jax
import jax.numpy as jnp
from jax import lax
from jax.experimental import pallas as pl
from jax.experimental.pallas import tpu as pltpu
from jax.experimental.pallas import tpu_sc as plsc

VOCAB = 100000
EMBED_DIM = 64
BATCH = 4096
SEQ_LEN = 200

NC = 2
NS = 16
NW = NC * NS

TOTAL = BATCH * SEQ_LEN
ROWS_PER_W = TOTAL // NW
CHUNK = 128
CHUNKS = ROWS_PER_W // CHUNK
NBUF = 4
GROUPS = CHUNKS // NBUF


@functools.partial(
    pl.kernel,
    out_type=jax.ShapeDtypeStruct((TOTAL, EMBED_DIM), jnp.float32),
    mesh=plsc.VectorSubcoreMesh(core_axis_name="c", subcore_axis_name="s"),
    compiler_params=pltpu.CompilerParams(use_tc_tiling_on_sc=False),
    scratch_types=(
        [pltpu.VMEM((CHUNKS, CHUNK), jnp.int32),
         pltpu.VMEM((NBUF, CHUNK, EMBED_DIM), jnp.float32)]
        + [pltpu.SemaphoreType.DMA] * (2 * NBUF)
    ),
)
def _embed_lookup(idx_hbm, table_hbm, out_hbm, idx_v, rows_v, *sems):
    gsem = sems[:NBUF]
    ssem = sems[NBUF:]
    wid = lax.axis_index("s") * NC + lax.axis_index("c")
    base = wid * ROWS_PER_W
    pltpu.sync_copy(idx_hbm.at[wid], idx_v)

    def gather(c, b):
        pltpu.async_copy(table_hbm.at[idx_v.at[c]], rows_v.at[b], gsem[b])

    def gather_wait(c, b):
        pltpu.make_async_copy(table_hbm.at[idx_v.at[c]], rows_v.at[b],
                              gsem[b]).wait()

    def store(c, b):
        pltpu.async_copy(rows_v.at[b],
                         out_hbm.at[pl.ds(base + c * CHUNK, CHUNK)], ssem[b])

    def store_wait(c, b):
        pltpu.make_async_copy(rows_v.at[b],
                              out_hbm.at[pl.ds(base + c * CHUNK, CHUNK)],
                              ssem[b]).wait()

    for b in range(NBUF):
        gather(b, b)
        gather_wait(b, b)
        store(b, b)

    def group(g, carry):
        for b in range(NBUF):
            c = g * NBUF + b
            store_wait(c - NBUF, b)
            gather(c, b)
            gather_wait(c, b)
            store(c, b)
        return carry

    lax.fori_loop(1, GROUPS, group, 0)

    for b in range(NBUF):
        c = (GROUPS - 1) * NBUF + b
        store_wait(c, b)


def kernel(x, word2vec_matrix):
    idx = x.reshape(NW, CHUNKS, CHUNK).astype(jnp.int32)
    out = _embed_lookup(idx, word2vec_matrix)
    return out.reshape(BATCH, SEQ_LEN, EMBED_DIM)


# speedup vs baseline: 4.2616x; 1.1280x over previous
import functools

import jax
import jax.numpy as jnp
from jax import lax
from jax.experimental import pallas as pl
from jax.experimental.pallas import tpu as pltpu
from jax.experimental.pallas import tpu_sc as plsc

VOCAB = 100000
EMBED_DIM = 64
BATCH = 4096
SEQ_LEN = 200

NC = 2
NS = 16
NW = NC * NS

TOTAL = BATCH * SEQ_LEN
ROWS_PER_W = TOTAL // NW
CHUNK = 128
CHUNKS = ROWS_PER_W // CHUNK
K = 5
GROUPS = CHUNKS // K


@functools.partial(
    pl.kernel,
    out_type=jax.ShapeDtypeStruct((TOTAL, EMBED_DIM), jnp.float32),
    mesh=plsc.VectorSubcoreMesh(core_axis_name="c", subcore_axis_name="s"),
    compiler_params=pltpu.CompilerParams(use_tc_tiling_on_sc=False),
    scratch_types=[
        pltpu.VMEM((CHUNKS, CHUNK), jnp.int32),
        pltpu.VMEM((2, K, CHUNK, EMBED_DIM), jnp.float32),
        pltpu.SemaphoreType.DMA,
        pltpu.SemaphoreType.DMA,
        pltpu.SemaphoreType.DMA,
    ],
)
def _embed_lookup(idx_hbm, table_hbm, out_hbm, idx_v, rows_v, gsem,
                  ssem0, ssem1):
    ssem = (ssem0, ssem1)
    wid = lax.axis_index("s") * NC + lax.axis_index("c")
    base = wid * ROWS_PER_W
    pltpu.sync_copy(idx_hbm.at[wid], idx_v)

    def gather_fire(g, h):
        for b in range(K):
            pltpu.async_copy(table_hbm.at[idx_v.at[g * K + b]],
                             rows_v.at[h, b], gsem)

    def gather_drain(g, h):
        for b in range(K):
            pltpu.make_async_copy(table_hbm.at[idx_v.at[g * K + b]],
                                  rows_v.at[h, b], gsem).wait()

    def store_fire(g, h):
        for b in range(K):
            c = g * K + b
            pltpu.async_copy(rows_v.at[h, b],
                             out_hbm.at[pl.ds(base + c * CHUNK, CHUNK)],
                             ssem[h])

    def store_drain(g, h):
        for b in range(K):
            c = g * K + b
            pltpu.make_async_copy(rows_v.at[h, b],
                                  out_hbm.at[pl.ds(base + c * CHUNK, CHUNK)],
                                  ssem[h]).wait()

    gather_fire(0, 0)
    gather_drain(0, 0)
    store_fire(0, 0)
    gather_fire(1, 1)
    gather_drain(1, 1)
    store_fire(1, 1)

    def group_pair(p, carry):
        for h in range(2):
            g = 2 * p + h
            store_drain(g - 2, h)
            gather_fire(g, h)
            gather_drain(g, h)
            store_fire(g, h)
        return carry

    lax.fori_loop(1, GROUPS // 2, group_pair, 0)

    store_drain(GROUPS - 2, 0)
    store_drain(GROUPS - 1, 1)


def kernel(x, word2vec_matrix):
    idx = x.reshape(NW, CHUNKS, CHUNK).astype(jnp.int32)
    out = _embed_lookup(idx, word2vec_matrix)
    return out.reshape(BATCH, SEQ_LEN, EMBED_DIM)
